# baseline (speedup 1.0000x reference)
import functools

import jax
import jax.numpy as jnp
from jax import lax
from jax.experimental import pallas as pl
from jax.experimental.pallas import tpu as pltpu
from jax.experimental.pallas import tpu_sc as plsc

_NUM_DENSE = 1024
_IFEAT = 64
_ICTX = 16
_HID = 128
_SPD = 32

_NC = 2
_NS = 16
_NW = _NC * _NS
_LANES = 16
_CH = 128


def _sc_gather_call(tokens3, if_pad, sp_pad, n, spread):
    pw = n // _NW
    nch = pw // _CH
    ng = nch // 2

    mesh = plsc.VectorSubcoreMesh(core_axis_name="c", subcore_axis_name="s")

    @functools.partial(
        pl.kernel,
        mesh=mesh,
        compiler_params=pltpu.CompilerParams(use_tc_tiling_on_sc=False),
        out_type=[
            jax.ShapeDtypeStruct((n, _IFEAT), jnp.float32),
            jax.ShapeDtypeStruct((n, _SPD), jnp.float32),
        ],
        scratch_types=[
            pltpu.VMEM((nch, _CH), jnp.int32),
            pltpu.VMEM((nch, _CH), jnp.int32),
            pltpu.VMEM((_CH, _IFEAT), jnp.float32),
            pltpu.VMEM((_CH, _SPD), jnp.float32),
            pltpu.VMEM((_CH, _IFEAT), jnp.float32),
            pltpu.VMEM((_CH, _SPD), jnp.float32),
            pltpu.SemaphoreType.DMA,
            pltpu.SemaphoreType.DMA,
            pltpu.SemaphoreType.DMA,
            pltpu.SemaphoreType.DMA,
        ],
    )
    def sc_gather(tok_hbm, if_hbm, sp_hbm,
                  if_out, sp_out,
                  tokall, spiall,
                  ifr0, spr0, ifr1, spr1,
                  gsem0, gsem1, ssem0, ssem1):
        wid = lax.axis_index("s") * _NC + lax.axis_index("c")
        base = wid * pw

        pltpu.sync_copy(tok_hbm.at[wid], tokall)

        def idx_chunk(c, carry):
            for i in range(_CH // _LANES):
                sl = pl.ds(i * _LANES, _LANES)
                t = tokall[c, sl]
                spiall[c, sl] = jnp.where(t > _NUM_DENSE, t - _NUM_DENSE,
                                          t + spread)
            return carry

        lax.fori_loop(0, nch, idx_chunk, 0)

        def fire(c, ifr, spr, gsem):
            pltpu.async_copy(if_hbm.at[tokall.at[c]], ifr, gsem)
            pltpu.async_copy(sp_hbm.at[spiall.at[c]], spr, gsem)

        def wait_gathers(c, ifr, spr, gsem):
            pltpu.make_async_copy(if_hbm.at[tokall.at[c]], ifr, gsem).wait()
            pltpu.make_async_copy(sp_hbm.at[spiall.at[c]], spr, gsem).wait()

        def fire_scatters(c, ifr, spr, ssem):
            off = base + c * _CH
            pltpu.async_copy(ifr, if_out.at[pl.ds(off, _CH)], ssem)
            pltpu.async_copy(spr, sp_out.at[pl.ds(off, _CH)], ssem)

        def wait_scatters(c, ifr, spr, ssem):
            off = base + c * _CH
            pltpu.make_async_copy(ifr, if_out.at[pl.ds(off, _CH)], ssem).wait()
            pltpu.make_async_copy(spr, sp_out.at[pl.ds(off, _CH)], ssem).wait()

        fire(0, ifr0, spr0, gsem0)
        fire(1, ifr1, spr1, gsem1)

        def group(g, carry):
            c0 = 2 * g
            c1 = c0 + 1
            wait_gathers(c0, ifr0, spr0, gsem0)
            fire_scatters(c0, ifr0, spr0, ssem0)
            wait_gathers(c1, ifr1, spr1, gsem1)
            fire_scatters(c1, ifr1, spr1, ssem1)

            @pl.when(g + 1 < ng)
            def _():
                wait_scatters(c0, ifr0, spr0, ssem0)
                fire(c0 + 2, ifr0, spr0, gsem0)
                wait_scatters(c1, ifr1, spr1, ssem1)
                fire(c1 + 2, ifr1, spr1, gsem1)

            return carry

        lax.fori_loop(0, ng, group, 0)

        wait_scatters(nch - 2, ifr0, spr0, ssem0)
        wait_scatters(nch - 1, ifr1, spr1, ssem1)

    return sc_gather(tokens3, if_pad, sp_pad)


def _tc_body(if_ref, ic_ref, sp_ref, tk_ref,
             wca_ref, wcb_ref, wsp_ref, d1_ref, be_ref, out_ref):
    dot = functools.partial(jnp.dot, preferred_element_type=jnp.float32)
    tok = tk_ref[...]
    acc = dot(if_ref[...], wca_ref[...]) + dot(ic_ref[...], wcb_ref[...])
    spw = dot(sp_ref[...], wsp_ref[...])
    dni = lax.rem(tok, d1_ref.shape[0])
    rows = lax.broadcasted_iota(jnp.int32, (1, d1_ref.shape[0]), 1)
    onehot = (dni == rows).astype(jnp.float32)
    dnc = jnp.dot(onehot, d1_ref[...],
                  preferred_element_type=jnp.float32)
    tv = jnp.where(tok > _NUM_DENSE, spw, dnc)
    v = acc + tv + be_ref[...]
    s = jnp.sum(v * v, axis=1, keepdims=True)
    nrm = jnp.maximum(jnp.sqrt(s), 1e-12)
    out_ref[...] = v / nrm


def _tc_call(if_g, ic2, sp_g, tk2, wca, wcb, wsp, d1, beff, n):
    t = 512
    g = n // t
    rtab = d1.shape[0]
    const = lambda shape: pl.BlockSpec(shape, lambda i: (0, 0))
    row = lambda d: pl.BlockSpec((t, d), lambda i: (i, 0))
    return pl.pallas_call(
        _tc_body,
        grid=(g,),
        in_specs=[
            row(_IFEAT), row(_ICTX), row(_SPD), row(1),
            const((_IFEAT, _HID)), const((_ICTX, _HID)),
            const((_SPD, _HID)), const((rtab, _HID)), const((1, _HID)),
        ],
        out_specs=row(_HID),
        out_shape=jax.ShapeDtypeStruct((n, _HID), jnp.float32),
    )(if_g, ic2, sp_g, tk2, wca, wcb, wsp, d1, beff)


def kernel(tokens, icontexts, ifeatures, dense_table, sparse_table,
           W_up, W_ac, b_ac, W_item, b_item):
    b, l = tokens.shape
    n = b * l
    pw = n // _NW
    nch = pw // _CH
    sparse_rows = sparse_table.shape[0]
    spread = sparse_rows - 1 - _NUM_DENSE

    w1 = W_item[:_HID]
    w2 = W_item[_HID:]
    wc = W_ac @ w2
    beff = (b_item + b_ac @ w2).reshape(1, _HID)
    wca = wc[:_IFEAT]
    wcb = wc[_IFEAT:]
    wsp = W_up @ w1
    d1 = dense_table @ w1

    tokens_flat = tokens.reshape(n).astype(jnp.int32)
    tokens3 = tokens_flat.reshape(_NW, nch, _CH)
    if_g, sp_g = _sc_gather_call(tokens3, ifeatures, sparse_table, n, spread)
    out = _tc_call(if_g, icontexts.reshape(n, _ICTX), sp_g,
                   tokens_flat.reshape(n, 1), wca, wcb, wsp, d1, beff, n)
    return out.reshape(b, l, _HID)


# speedup vs baseline: 1.0828x; 1.0828x over previous
import functools

import jax
import jax.numpy as jnp
from jax import lax
from jax.experimental import pallas as pl
from jax.experimental.pallas import tpu as pltpu
from jax.experimental.pallas import tpu_sc as plsc

_NUM_DENSE = 1024
_IFEAT = 64
_ICTX = 16
_HID = 128
_SPD = 32

_NC = 2
_NS = 16
_NW = _NC * _NS
_LANES = 16
_CH = 128


def _sc_gather_call(tokens3, if_pad, sp_pad, n, spread):
    pw = n // _NW
    nch = pw // _CH
    ng = nch // 2

    mesh = plsc.VectorSubcoreMesh(core_axis_name="c", subcore_axis_name="s")

    @functools.partial(
        pl.kernel,
        mesh=mesh,
        compiler_params=pltpu.CompilerParams(use_tc_tiling_on_sc=False),
        out_type=jax.ShapeDtypeStruct((n, _HID), jnp.float32),
        scratch_types=[
            pltpu.VMEM((nch, _CH), jnp.int32),
            pltpu.VMEM((nch, _CH), jnp.int32),
            pltpu.VMEM((_CH, _IFEAT), jnp.float32),
            pltpu.VMEM((_CH, _SPD), jnp.float32),
            pltpu.VMEM((_CH, _IFEAT), jnp.float32),
            pltpu.VMEM((_CH, _SPD), jnp.float32),
            pltpu.SemaphoreType.DMA,
            pltpu.SemaphoreType.DMA,
            pltpu.SemaphoreType.DMA,
            pltpu.SemaphoreType.DMA,
        ],
    )
    def sc_gather(tok_hbm, if_hbm, sp_hbm,
                  pk_out,
                  tokall, spiall,
                  ifr0, spr0, ifr1, spr1,
                  gsem0, gsem1, ssem0, ssem1):
        wid = lax.axis_index("s") * _NC + lax.axis_index("c")
        base = wid * pw

        pltpu.sync_copy(tok_hbm.at[wid], tokall)

        def idx_chunk(c, carry):
            for i in range(_CH // _LANES):
                sl = pl.ds(i * _LANES, _LANES)
                t = tokall[c, sl]
                spiall[c, sl] = jnp.where(t > _NUM_DENSE, t - _NUM_DENSE,
                                          t + spread)
            return carry

        lax.fori_loop(0, nch, idx_chunk, 0)

        def fire(c, ifr, spr, gsem):
            pltpu.async_copy(if_hbm.at[tokall.at[c]], ifr, gsem)
            pltpu.async_copy(sp_hbm.at[spiall.at[c]], spr, gsem)

        def wait_gathers(c, ifr, spr, gsem):
            pltpu.make_async_copy(if_hbm.at[tokall.at[c]], ifr, gsem).wait()
            pltpu.make_async_copy(sp_hbm.at[spiall.at[c]], spr, gsem).wait()

        def fire_scatters(c, ifr, spr, ssem):
            off = base + c * _CH
            rows = pl.ds(off, _CH)
            pltpu.async_copy(ifr, pk_out.at[rows, pl.ds(0, _IFEAT)], ssem)
            pltpu.async_copy(spr, pk_out.at[rows, pl.ds(_IFEAT, _SPD)], ssem)
            pltpu.async_copy(spr, pk_out.at[rows, pl.ds(_IFEAT + _SPD, _SPD)],
                             ssem)

        def wait_scatters(c, ifr, spr, ssem):
            off = base + c * _CH
            rows = pl.ds(off, _CH)
            pltpu.make_async_copy(ifr, pk_out.at[rows, pl.ds(0, _IFEAT)],
                                  ssem).wait()
            pltpu.make_async_copy(spr, pk_out.at[rows, pl.ds(_IFEAT, _SPD)],
                                  ssem).wait()
            pltpu.make_async_copy(spr, pk_out.at[rows,
                                                 pl.ds(_IFEAT + _SPD, _SPD)],
                                  ssem).wait()

        fire(0, ifr0, spr0, gsem0)
        fire(1, ifr1, spr1, gsem1)

        def group(g, carry):
            c0 = 2 * g
            c1 = c0 + 1
            wait_gathers(c0, ifr0, spr0, gsem0)
            fire_scatters(c0, ifr0, spr0, ssem0)
            wait_gathers(c1, ifr1, spr1, gsem1)
            fire_scatters(c1, ifr1, spr1, ssem1)

            @pl.when(g + 1 < ng)
            def _():
                wait_scatters(c0, ifr0, spr0, ssem0)
                fire(c0 + 2, ifr0, spr0, gsem0)
                wait_scatters(c1, ifr1, spr1, ssem1)
                fire(c1 + 2, ifr1, spr1, gsem1)

            return carry

        lax.fori_loop(0, ng, group, 0)

        wait_scatters(nch - 2, ifr0, spr0, ssem0)
        wait_scatters(nch - 1, ifr1, spr1, ssem1)

    return sc_gather(tokens3, if_pad, sp_pad)


def _tc_body(pk_ref, ic_ref, tk_ref,
             wif_ref, wcb_ref, wspx_ref, d1_ref, be_ref, out_ref):
    dot = functools.partial(jnp.dot, preferred_element_type=jnp.float32)
    tok = tk_ref[...]
    pk = pk_ref[...]
    acc = dot(pk, wif_ref[...]) + dot(ic_ref[...], wcb_ref[...])
    spw = dot(pk, wspx_ref[...])
    dni = lax.rem(tok, d1_ref.shape[0])
    rows = lax.broadcasted_iota(jnp.int32, (1, d1_ref.shape[0]), 1)
    onehot = (dni == rows).astype(jnp.float32)
    dnc = jnp.dot(onehot, d1_ref[...],
                  preferred_element_type=jnp.float32)
    tv = jnp.where(tok > _NUM_DENSE, spw, dnc)
    v = acc + tv + be_ref[...]
    s = jnp.sum(v * v, axis=1, keepdims=True)
    nrm = jnp.maximum(jnp.sqrt(s), 1e-12)
    out_ref[...] = v / nrm


def _tc_call(pk_g, ic2, tk2, wif, wcb, wspx, d1, beff, n):
    t = 512
    g = n // t
    rtab = d1.shape[0]
    const = lambda shape: pl.BlockSpec(shape, lambda i: (0, 0))
    row = lambda d: pl.BlockSpec((t, d), lambda i: (i, 0))
    return pl.pallas_call(
        _tc_body,
        grid=(g,),
        in_specs=[
            row(_HID), row(_ICTX), row(1),
            const((_HID, _HID)), const((_ICTX, _HID)),
            const((_HID, _HID)), const((rtab, _HID)), const((1, _HID)),
        ],
        out_specs=row(_HID),
        out_shape=jax.ShapeDtypeStruct((n, _HID), jnp.float32),
    )(pk_g, ic2, tk2, wif, wcb, wspx, d1, beff)


def kernel(tokens, icontexts, ifeatures, dense_table, sparse_table,
           W_up, W_ac, b_ac, W_item, b_item):
    b, l = tokens.shape
    n = b * l
    pw = n // _NW
    nch = pw // _CH
    sparse_rows = sparse_table.shape[0]
    spread = sparse_rows - 1 - _NUM_DENSE

    w1 = W_item[:_HID]
    w2 = W_item[_HID:]
    wc = W_ac @ w2
    beff = (b_item + b_ac @ w2).reshape(1, _HID)
    wcb = wc[_IFEAT:]
    wif = jnp.pad(wc[:_IFEAT], ((0, _HID - _IFEAT), (0, 0)))
    wspx = jnp.pad(W_up @ w1, ((_IFEAT, _HID - _IFEAT - _SPD), (0, 0)))
    d1 = dense_table @ w1

    tokens_flat = tokens.reshape(n).astype(jnp.int32)
    tokens3 = tokens_flat.reshape(_NW, nch, _CH)
    pk_g = _sc_gather_call(tokens3, ifeatures, sparse_table, n, spread)
    out = _tc_call(pk_g, icontexts.reshape(n, _ICTX),
                   tokens_flat.reshape(n, 1), wif, wcb, wspx, d1, beff, n)
    return out.reshape(b, l, _HID)


# speedup vs baseline: 1.0979x; 1.0139x over previous
import functools

import jax
import jax.numpy as jnp
from jax import lax
from jax.experimental import pallas as pl
from jax.experimental.pallas import tpu as pltpu
from jax.experimental.pallas import tpu_sc as plsc

_NUM_DENSE = 1024
_IFEAT = 64
_ICTX = 16
_HID = 128
_SPD = 32

_NC = 2
_NS = 16
_NW = _NC * _NS
_LANES = 16
_CH = 128


def _sc_gather_call(tokens3, if_pad, sp_pad, ic2, n, spread):
    pw = n // _NW
    nch = pw // _CH
    ng = nch // 2

    mesh = plsc.VectorSubcoreMesh(core_axis_name="c", subcore_axis_name="s")

    @functools.partial(
        pl.kernel,
        mesh=mesh,
        compiler_params=pltpu.CompilerParams(use_tc_tiling_on_sc=False),
        out_type=jax.ShapeDtypeStruct((n, _HID), jnp.float32),
        scratch_types=[
            pltpu.VMEM((nch, _CH), jnp.int32),
            pltpu.VMEM((nch, _CH), jnp.int32),
            pltpu.VMEM((_CH, _IFEAT), jnp.float32),
            pltpu.VMEM((_CH, _SPD), jnp.float32),
            pltpu.VMEM((_CH, _ICTX), jnp.float32),
            pltpu.VMEM((_CH, _IFEAT), jnp.float32),
            pltpu.VMEM((_CH, _SPD), jnp.float32),
            pltpu.VMEM((_CH, _ICTX), jnp.float32),
            pltpu.SemaphoreType.DMA,
            pltpu.SemaphoreType.DMA,
            pltpu.SemaphoreType.DMA,
            pltpu.SemaphoreType.DMA,
        ],
    )
    def sc_gather(tok_hbm, if_hbm, sp_hbm, ic_hbm,
                  pk_out,
                  tokall, spiall,
                  ifr0, spr0, icr0, ifr1, spr1, icr1,
                  gsem0, gsem1, ssem0, ssem1):
        wid = lax.axis_index("s") * _NC + lax.axis_index("c")
        base = wid * pw

        pltpu.sync_copy(tok_hbm.at[wid], tokall)

        def idx_chunk(c, carry):
            for i in range(_CH // _LANES):
                sl = pl.ds(i * _LANES, _LANES)
                t = tokall[c, sl]
                spiall[c, sl] = jnp.where(t > _NUM_DENSE, t - _NUM_DENSE,
                                          t + spread)
            return carry

        lax.fori_loop(0, nch, idx_chunk, 0)

        def fire(c, ifr, spr, icr, gsem):
            off = base + c * _CH
            pltpu.async_copy(if_hbm.at[tokall.at[c]], ifr, gsem)
            pltpu.async_copy(sp_hbm.at[spiall.at[c]], spr, gsem)
            pltpu.async_copy(ic_hbm.at[pl.ds(off, _CH)], icr, gsem)

        def wait_gathers(c, ifr, spr, icr, gsem):
            off = base + c * _CH
            pltpu.make_async_copy(if_hbm.at[tokall.at[c]], ifr, gsem).wait()
            pltpu.make_async_copy(sp_hbm.at[spiall.at[c]], spr, gsem).wait()
            pltpu.make_async_copy(ic_hbm.at[pl.ds(off, _CH)], icr, gsem).wait()

        def fire_scatters(c, ifr, spr, icr, ssem):
            off = base + c * _CH
            rows = pl.ds(off, _CH)
            pltpu.async_copy(ifr, pk_out.at[rows, pl.ds(0, _IFEAT)], ssem)
            pltpu.async_copy(spr, pk_out.at[rows, pl.ds(_IFEAT, _SPD)], ssem)
            pltpu.async_copy(icr, pk_out.at[rows, pl.ds(96, _ICTX)], ssem)
            pltpu.async_copy(icr, pk_out.at[rows, pl.ds(112, _ICTX)], ssem)

        def wait_scatters(c, ifr, spr, icr, ssem):
            off = base + c * _CH
            rows = pl.ds(off, _CH)
            pltpu.make_async_copy(ifr, pk_out.at[rows, pl.ds(0, _IFEAT)],
                                  ssem).wait()
            pltpu.make_async_copy(spr, pk_out.at[rows, pl.ds(_IFEAT, _SPD)],
                                  ssem).wait()
            pltpu.make_async_copy(icr, pk_out.at[rows, pl.ds(96, _ICTX)],
                                  ssem).wait()
            pltpu.make_async_copy(icr, pk_out.at[rows, pl.ds(112, _ICTX)],
                                  ssem).wait()

        fire(0, ifr0, spr0, icr0, gsem0)
        fire(1, ifr1, spr1, icr1, gsem1)

        def group(g, carry):
            c0 = 2 * g
            c1 = c0 + 1
            wait_gathers(c0, ifr0, spr0, icr0, gsem0)
            fire_scatters(c0, ifr0, spr0, icr0, ssem0)
            wait_gathers(c1, ifr1, spr1, icr1, gsem1)
            fire_scatters(c1, ifr1, spr1, icr1, ssem1)

            @pl.when(g + 1 < ng)
            def _():
                wait_scatters(c0, ifr0, spr0, icr0, ssem0)
                fire(c0 + 2, ifr0, spr0, icr0, gsem0)
                wait_scatters(c1, ifr1, spr1, icr1, ssem1)
                fire(c1 + 2, ifr1, spr1, icr1, gsem1)

            return carry

        lax.fori_loop(0, ng, group, 0)

        wait_scatters(nch - 2, ifr0, spr0, icr0, ssem0)
        wait_scatters(nch - 1, ifr1, spr1, icr1, ssem1)

    return sc_gather(tokens3, if_pad, sp_pad, ic2)


def _tc_body(pk_ref, tk_ref,
             wif_ref, wspx_ref, d1_ref, be_ref, out_ref):
    dot = functools.partial(jnp.dot, preferred_element_type=jnp.float32)
    tok = tk_ref[...]
    pk = pk_ref[...]
    acc = dot(pk, wif_ref[...])
    spw = dot(pk, wspx_ref[...])
    dni = jnp.where(tok > _NUM_DENSE, -1, tok)
    rows = lax.broadcasted_iota(jnp.int32, (1, d1_ref.shape[0]), 1)
    onehot = (dni == rows).astype(jnp.float32)
    dnc = jnp.dot(onehot, d1_ref[...],
                  preferred_element_type=jnp.float32)
    tv = jnp.where(tok > _NUM_DENSE, spw, dnc)
    v = acc + tv + be_ref[...]
    s = jnp.sum(v * v, axis=1, keepdims=True)
    nrm = jnp.maximum(jnp.sqrt(s), 1e-12)
    out_ref[...] = v / nrm


def _tc_call(pk_g, tk2, wif, wspx, d1, beff, n):
    t = 512
    g = n // t
    rtab = d1.shape[0]
    const = lambda shape: pl.BlockSpec(shape, lambda i: (0, 0))
    row = lambda d: pl.BlockSpec((t, d), lambda i: (i, 0))
    return pl.pallas_call(
        _tc_body,
        grid=(g,),
        in_specs=[
            row(_HID), row(1),
            const((_HID, _HID)),
            const((_HID, _HID)), const((rtab, _HID)), const((1, _HID)),
        ],
        out_specs=row(_HID),
        out_shape=jax.ShapeDtypeStruct((n, _HID), jnp.float32),
    )(pk_g, tk2, wif, wspx, d1, beff)


def kernel(tokens, icontexts, ifeatures, dense_table, sparse_table,
           W_up, W_ac, b_ac, W_item, b_item):
    b, l = tokens.shape
    n = b * l
    pw = n // _NW
    nch = pw // _CH
    sparse_rows = sparse_table.shape[0]
    spread = sparse_rows - 1 - _NUM_DENSE

    w1 = W_item[:_HID]
    w2 = W_item[_HID:]
    wc = W_ac @ w2
    beff = (b_item + b_ac @ w2).reshape(1, _HID)
    z32 = jnp.zeros((_SPD, _HID), jnp.float32)
    z16 = jnp.zeros((_ICTX, _HID), jnp.float32)
    wif = jnp.concatenate([wc[:_IFEAT], z32, wc[_IFEAT:], z16], axis=0)
    wspx = jnp.pad(W_up @ w1, ((_IFEAT, _HID - _IFEAT - _SPD), (0, 0)))
    d1 = dense_table @ w1

    tokens_flat = tokens.reshape(n).astype(jnp.int32)
    tokens3 = tokens_flat.reshape(_NW, nch, _CH)
    pk_g = _sc_gather_call(tokens3, ifeatures, sparse_table,
                           icontexts.reshape(n, _ICTX), n, spread)
    out = _tc_call(pk_g, tokens_flat.reshape(n, 1), wif, wspx, d1, beff, n)
    return out.reshape(b, l, _HID)
